# Initial kernel scaffold; baseline (speedup 1.0000x reference)
import functools

import jax
import jax.numpy as jnp
from jax import lax
from jax.experimental import pallas as pl
from jax.experimental.pallas import tpu as pltpu
from jax.experimental.pallas import tpu_sc as plsc

N = 10000
N_PAD = 10240
E = 320000
IN_F = 128
HID = 128
CLS = 64

NC = 2
NS = 16
NW = NC * NS
E_PER_W = E // NW
K = 80
NCHUNK = E_PER_W // K
RPS = N_PAD // NS

BLK = 1024


def _sc_degree(dst):
    mesh = plsc.VectorSubcoreMesh(core_axis_name="c", subcore_axis_name="s")
    e0 = jnp.zeros((K, 16), jnp.float32).at[:, 0].set(1.0)
    zrows = jnp.zeros((RPS, 16), jnp.float32)

    @functools.partial(
        pl.kernel,
        out_type=jax.ShapeDtypeStruct((NC, N_PAD, 16), jnp.float32),
        mesh=mesh,
        scratch_types=[
            pltpu.VMEM((K,), jnp.int32),
            pltpu.VMEM((K, 16), jnp.float32),
            pltpu.VMEM_SHARED((N_PAD, 16), jnp.float32),
        ],
    )
    def k(dst_hbm, e0_hbm, z_hbm, out_hbm, dst_v, ones_v, acc):
        c = lax.axis_index("c")
        s = lax.axis_index("s")
        w = c * NS + s
        pltpu.sync_copy(z_hbm, acc.at[pl.ds(s * RPS, RPS)])
        pltpu.sync_copy(e0_hbm, ones_v)
        plsc.subcore_barrier()

        @pl.loop(0, NCHUNK)
        def _(i):
            base = w * E_PER_W + i * K
            pltpu.sync_copy(dst_hbm.at[pl.ds(base, K)], dst_v)
            pltpu.sync_copy(ones_v, acc.at[dst_v], add=True)

        plsc.subcore_barrier()
        pltpu.sync_copy(acc.at[pl.ds(s * RPS, RPS)],
                        out_hbm.at[c, pl.ds(s * RPS, RPS)])

    return k(dst, e0, zrows)


def _sc_scatter(table, src, dst, d):
    mesh = plsc.VectorSubcoreMesh(core_axis_name="c", subcore_axis_name="s")
    zrows = jnp.zeros((RPS, d), jnp.float32)

    @functools.partial(
        pl.kernel,
        out_type=jax.ShapeDtypeStruct((NC, N_PAD, d), jnp.float32),
        mesh=mesh,
        scratch_types=[
            pltpu.VMEM((K,), jnp.int32),
            pltpu.VMEM((K,), jnp.int32),
            pltpu.VMEM((K, d), jnp.float32),
            pltpu.VMEM_SHARED((N_PAD, d), jnp.float32),
            pltpu.SemaphoreType.DMA,
        ],
    )
    def k(table_hbm, src_hbm, dst_hbm, z_hbm, out_hbm,
          src_v, dst_v, rows_v, acc, sem):
        c = lax.axis_index("c")
        s = lax.axis_index("s")
        w = c * NS + s
        pltpu.sync_copy(z_hbm, acc.at[pl.ds(s * RPS, RPS)])
        plsc.subcore_barrier()

        @pl.loop(0, NCHUNK)
        def _(i):
            base = w * E_PER_W + i * K
            pltpu.sync_copy(src_hbm.at[pl.ds(base, K)], src_v)
            pltpu.async_copy(table_hbm.at[src_v], rows_v, sem).wait()
            pltpu.sync_copy(dst_hbm.at[pl.ds(base, K)], dst_v)
            pltpu.sync_copy(rows_v, acc.at[dst_v], add=True)

        plsc.subcore_barrier()
        pltpu.sync_copy(acc.at[pl.ds(s * RPS, RPS)],
                        out_hbm.at[c, pl.ds(s * RPS, RPS)])

    return k(table, src, dst, zrows)


def _tc_layer1(deg_p, x_pad, w1):
    def body(degp_ref, x_ref, w_ref, g_ref, dinv_ref):
        deg = degp_ref[0, :, 0] + degp_ref[1, :, 0] + 1.0
        dinv = lax.rsqrt(deg)
        h = jnp.dot(x_ref[...], w_ref[...], preferred_element_type=jnp.float32)
        g_ref[...] = h * dinv[:, None]
        dinv_ref[...] = dinv

    return pl.pallas_call(
        body,
        grid=(N_PAD // BLK,),
        in_specs=[
            pl.BlockSpec((NC, BLK, 16), lambda i: (0, i, 0)),
            pl.BlockSpec((BLK, IN_F), lambda i: (i, 0)),
            pl.BlockSpec((IN_F, HID), lambda i: (0, 0)),
        ],
        out_specs=[
            pl.BlockSpec((BLK, HID), lambda i: (i, 0)),
            pl.BlockSpec((BLK,), lambda i: (i,)),
        ],
        out_shape=[
            jax.ShapeDtypeStruct((N_PAD, HID), jnp.float32),
            jax.ShapeDtypeStruct((N_PAD,), jnp.float32),
        ],
    )(deg_p, x_pad, w1)


def _tc_layer2(s1_p, g1, dinv, b1, w2):
    def body(sp_ref, g1_ref, dinv_ref, b1_ref, w_ref, g2_ref):
        dinv = dinv_ref[...]
        z = (sp_ref[0] + sp_ref[1] + g1_ref[...]) * dinv[:, None] + b1_ref[...]
        z = jnp.maximum(z, 0.0)
        h = jnp.dot(z, w_ref[...], preferred_element_type=jnp.float32)
        g2_ref[...] = h * dinv[:, None]

    return pl.pallas_call(
        body,
        grid=(N_PAD // BLK,),
        in_specs=[
            pl.BlockSpec((NC, BLK, HID), lambda i: (0, i, 0)),
            pl.BlockSpec((BLK, HID), lambda i: (i, 0)),
            pl.BlockSpec((BLK,), lambda i: (i,)),
            pl.BlockSpec((HID,), lambda i: (0,)),
            pl.BlockSpec((HID, CLS), lambda i: (0, 0)),
        ],
        out_specs=pl.BlockSpec((BLK, CLS), lambda i: (i, 0)),
        out_shape=jax.ShapeDtypeStruct((N_PAD, CLS), jnp.float32),
    )(s1_p, g1, dinv, b1, w2)


def _tc_out(s2_p, g2, dinv, b2):
    def body(sp_ref, g2_ref, dinv_ref, b2_ref, o_ref):
        o_ref[...] = ((sp_ref[0] + sp_ref[1] + g2_ref[...])
                      * dinv_ref[...][:, None] + b2_ref[...])

    return pl.pallas_call(
        body,
        grid=(N_PAD // BLK,),
        in_specs=[
            pl.BlockSpec((NC, BLK, CLS), lambda i: (0, i, 0)),
            pl.BlockSpec((BLK, CLS), lambda i: (i, 0)),
            pl.BlockSpec((BLK,), lambda i: (i,)),
            pl.BlockSpec((CLS,), lambda i: (0,)),
        ],
        out_specs=pl.BlockSpec((BLK, CLS), lambda i: (i, 0)),
        out_shape=jax.ShapeDtypeStruct((N_PAD, CLS), jnp.float32),
    )(s2_p, g2, dinv, b2)


def kernel(x, edge_index, W1, b1, W2, b2):
    ei = edge_index.astype(jnp.int32)
    src, dst = ei[0], ei[1]
    x_pad = jnp.pad(x, ((0, N_PAD - N), (0, 0)))

    deg_p = _sc_degree(dst)
    g1, dinv = _tc_layer1(deg_p, x_pad, W1)
    s1_p = _sc_scatter(g1, src, dst, HID)
    g2 = _tc_layer2(s1_p, g1, dinv, b1, W2)
    s2_p = _sc_scatter(g2, src, dst, CLS)
    out = _tc_out(s2_p, g2, dinv, b2)
    return out[:N]


# speedup vs baseline: 12.0823x; 12.0823x over previous
import functools

import jax
import jax.numpy as jnp
from jax import lax
from jax.experimental import pallas as pl
from jax.experimental.pallas import tpu as pltpu
from jax.experimental.pallas import tpu_sc as plsc

N = 10000
N_PAD = 10240
E = 320000
IN_F = 128
HID = 128
CLS = 64

NC = 2
NS = 16
NW = NC * NS
E_PER_W = E // NW
K = 80
NCHUNK = E_PER_W // K
RPS = N_PAD // NS

BLK = 1024


def _sc_degree(dst):
    mesh = plsc.VectorSubcoreMesh(core_axis_name="c", subcore_axis_name="s")
    e0 = jnp.zeros((K, HID), jnp.float32).at[:, 0].set(1.0)
    zrows = jnp.zeros((RPS, HID), jnp.float32)

    @functools.partial(
        pl.kernel,
        out_type=jax.ShapeDtypeStruct((NC, N_PAD, HID), jnp.float32),
        mesh=mesh,
        scratch_types=[
            pltpu.VMEM((K,), jnp.int32),
            pltpu.VMEM((K, HID), jnp.float32),
            pltpu.VMEM_SHARED((N_PAD, HID), jnp.float32),
        ],
    )
    def k(dst_hbm, e0_hbm, z_hbm, out_hbm, dst_v, ones_v, acc):
        c = lax.axis_index("c")
        s = lax.axis_index("s")
        w = c * NS + s
        pltpu.sync_copy(z_hbm, acc.at[pl.ds(s * RPS, RPS)])
        pltpu.sync_copy(e0_hbm, ones_v)
        plsc.subcore_barrier()

        @pl.loop(0, NCHUNK)
        def _(i):
            base = w * E_PER_W + i * K
            pltpu.sync_copy(dst_hbm.at[pl.ds(base, K)], dst_v)
            pltpu.sync_copy(ones_v, acc.at[dst_v], add=True)

        plsc.subcore_barrier()
        pltpu.sync_copy(acc.at[pl.ds(s * RPS, RPS)],
                        out_hbm.at[c, pl.ds(s * RPS, RPS)])

    return k(dst, e0, zrows)


def _sc_scatter(table, src, dst, d):
    mesh = plsc.VectorSubcoreMesh(core_axis_name="c", subcore_axis_name="s")
    zrows = jnp.zeros((RPS, d), jnp.float32)

    @functools.partial(
        pl.kernel,
        out_type=jax.ShapeDtypeStruct((NC, N_PAD, d), jnp.float32),
        mesh=mesh,
        scratch_types=[
            pltpu.VMEM((K,), jnp.int32),
            pltpu.VMEM((K,), jnp.int32),
            pltpu.VMEM((K, d), jnp.float32),
            pltpu.VMEM_SHARED((N_PAD, d), jnp.float32),
            pltpu.SemaphoreType.DMA,
        ],
    )
    def k(table_hbm, src_hbm, dst_hbm, z_hbm, out_hbm,
          src_v, dst_v, rows_v, acc, sem):
        c = lax.axis_index("c")
        s = lax.axis_index("s")
        w = c * NS + s
        pltpu.sync_copy(z_hbm, acc.at[pl.ds(s * RPS, RPS)])
        plsc.subcore_barrier()

        @pl.loop(0, NCHUNK)
        def _(i):
            base = w * E_PER_W + i * K
            pltpu.sync_copy(src_hbm.at[pl.ds(base, K)], src_v)
            pltpu.async_copy(table_hbm.at[src_v], rows_v, sem).wait()
            pltpu.sync_copy(dst_hbm.at[pl.ds(base, K)], dst_v)
            pltpu.sync_copy(rows_v, acc.at[dst_v], add=True)

        plsc.subcore_barrier()
        pltpu.sync_copy(acc.at[pl.ds(s * RPS, RPS)],
                        out_hbm.at[c, pl.ds(s * RPS, RPS)])

    return k(table, src, dst, zrows)


def _tc_layer1(deg_p, x_pad, w1):
    def body(degp_ref, x_ref, w_ref, g_ref, dinv_ref):
        deg = degp_ref[0, :, 0] + degp_ref[1, :, 0] + 1.0
        dinv = lax.rsqrt(deg)
        h = jnp.dot(x_ref[...], w_ref[...], preferred_element_type=jnp.float32)
        g_ref[...] = h * dinv[:, None]
        dinv_ref[...] = dinv

    return pl.pallas_call(
        body,
        grid=(N_PAD // BLK,),
        in_specs=[
            pl.BlockSpec((NC, BLK, HID), lambda i: (0, i, 0)),
            pl.BlockSpec((BLK, IN_F), lambda i: (i, 0)),
            pl.BlockSpec((IN_F, HID), lambda i: (0, 0)),
        ],
        out_specs=[
            pl.BlockSpec((BLK, HID), lambda i: (i, 0)),
            pl.BlockSpec((BLK,), lambda i: (i,)),
        ],
        out_shape=[
            jax.ShapeDtypeStruct((N_PAD, HID), jnp.float32),
            jax.ShapeDtypeStruct((N_PAD,), jnp.float32),
        ],
    )(deg_p, x_pad, w1)


def _tc_layer2(s1_p, g1, dinv, b1, w2):
    def body(sp_ref, g1_ref, dinv_ref, b1_ref, w_ref, g2_ref):
        dinv = dinv_ref[...]
        z = (sp_ref[0] + sp_ref[1] + g1_ref[...]) * dinv[:, None] + b1_ref[...]
        z = jnp.maximum(z, 0.0)
        h = jnp.dot(z, w_ref[...], preferred_element_type=jnp.float32)
        g2_ref[...] = h * dinv[:, None]

    return pl.pallas_call(
        body,
        grid=(N_PAD // BLK,),
        in_specs=[
            pl.BlockSpec((NC, BLK, HID), lambda i: (0, i, 0)),
            pl.BlockSpec((BLK, HID), lambda i: (i, 0)),
            pl.BlockSpec((BLK,), lambda i: (i,)),
            pl.BlockSpec((HID,), lambda i: (0,)),
            pl.BlockSpec((HID, HID), lambda i: (0, 0)),
        ],
        out_specs=pl.BlockSpec((BLK, HID), lambda i: (i, 0)),
        out_shape=jax.ShapeDtypeStruct((N_PAD, HID), jnp.float32),
    )(s1_p, g1, dinv, b1, w2)


def _tc_out(s2_p, g2, dinv, b2):
    def body(sp_ref, g2_ref, dinv_ref, b2_ref, o_ref):
        o_ref[...] = ((sp_ref[0] + sp_ref[1] + g2_ref[...])
                      * dinv_ref[...][:, None] + b2_ref[...])

    return pl.pallas_call(
        body,
        grid=(N_PAD // BLK,),
        in_specs=[
            pl.BlockSpec((NC, BLK, HID), lambda i: (0, i, 0)),
            pl.BlockSpec((BLK, HID), lambda i: (i, 0)),
            pl.BlockSpec((BLK,), lambda i: (i,)),
            pl.BlockSpec((HID,), lambda i: (0,)),
        ],
        out_specs=pl.BlockSpec((BLK, HID), lambda i: (i, 0)),
        out_shape=jax.ShapeDtypeStruct((N_PAD, HID), jnp.float32),
    )(s2_p, g2, dinv, b2)


def kernel(x, edge_index, W1, b1, W2, b2):
    ei = edge_index.astype(jnp.int32)
    src, dst = ei[0], ei[1]
    x_pad = jnp.pad(x, ((0, N_PAD - N), (0, 0)))
    w2_pad = jnp.pad(W2, ((0, 0), (0, HID - CLS)))
    b2_pad = jnp.pad(b2, ((0, HID - CLS),))

    deg_p = _sc_degree(dst)
    g1, dinv = _tc_layer1(deg_p, x_pad, W1)
    s1_p = _sc_scatter(g1, src, dst, HID)
    g2 = _tc_layer2(s1_p, g1, dinv, b1, w2_pad)
    s2_p = _sc_scatter(g2, src, dst, HID)
    out = _tc_out(s2_p, g2, dinv, b2_pad)
    return out[:N, :CLS]
